# baseline (device time: 25970 ns/iter reference)
import jax
import jax.numpy as jnp
from jax import lax
from jax.experimental import pallas as pl
from jax.experimental.pallas import tpu as pltpu

N_DEV = 4
M_CHUNK = 512
MH = M_CHUNK // 2
D = 512
EPS = 1e-6


def kernel(partial, gamma):
    gamma2d = gamma.reshape(1, D)

    def body(x_ref, g_ref, out_ref, rp1, rp2, stage,
             p1_send, p1_recv, p2_send, p2_recv):
        my = lax.axis_index("i")
        pa = my ^ 1
        pb = 3 - my

        barrier_sem = pltpu.get_barrier_semaphore()
        for nbr in (pa, pb):
            pl.semaphore_signal(
                barrier_sem, inc=1,
                device_id=(nbr,), device_id_type=pl.DeviceIdType.MESH,
            )
        pl.semaphore_wait(barrier_sem, 2)

        def sub_at(c, s):
            return x_ref.at[0, pl.ds(c * M_CHUNK + s * MH, MH), :]

        def sub(c, s):
            return x_ref[0, pl.ds(c * M_CHUNK + s * MH, MH), :]

        def p1_rdma(slot, c, s, target):
            return pltpu.make_async_remote_copy(
                src_ref=sub_at(c, s),
                dst_ref=rp1.at[slot],
                send_sem=p1_send.at[slot],
                recv_sem=p1_recv.at[slot],
                device_id=(target,),
                device_id_type=pl.DeviceIdType.MESH,
            )

        rB = p1_rdma(0, 3 - pa, 0, pa)
        rD = p1_rdma(2, pb ^ 1, 1, pb)
        rA = p1_rdma(1, pa, 0, pa)
        rC = p1_rdma(3, pb, 1, pb)
        rB.start()
        rD.start()
        rA.start()
        rC.start()

        def p2_rdma(slot, target):
            return pltpu.make_async_remote_copy(
                src_ref=stage.at[slot],
                dst_ref=rp2.at[slot],
                send_sem=p2_send.at[slot],
                recv_sem=p2_recv.at[slot],
                device_id=(target,),
                device_id_type=pl.DeviceIdType.MESH,
            )

        rB.wait_recv()
        stage[0] = sub(3 - my, 0) + rp1[0]
        r3 = p2_rdma(0, pb)
        r3.start()

        rD.wait_recv()
        stage[1] = sub(my ^ 1, 1) + rp1[2]
        r4 = p2_rdma(1, pa)
        r4.start()

        rA.wait_recv()
        a0 = sub(my, 0) + rp1[1]
        rC.wait_recv()
        a1 = sub(my, 1) + rp1[3]

        def norm(y):
            ms = jnp.mean(y * y, axis=-1, keepdims=True)
            return y * lax.rsqrt(ms + EPS) * g_ref[...]

        r3.wait_recv()
        out_ref[0:MH, :] = norm(a0 + rp2[0])
        r4.wait_recv()
        out_ref[MH:M_CHUNK, :] = norm(a1 + rp2[1])

        for r in (rB, rD, rA, rC, r3, r4):
            r.wait_send()

    return pl.pallas_call(
        body,
        out_shape=jax.ShapeDtypeStruct((M_CHUNK, D), jnp.float32),
        in_specs=[
            pl.BlockSpec(memory_space=pltpu.VMEM),
            pl.BlockSpec(memory_space=pltpu.VMEM),
        ],
        out_specs=pl.BlockSpec(memory_space=pltpu.VMEM),
        scratch_shapes=[
            pltpu.VMEM((4, MH, D), jnp.float32),
            pltpu.VMEM((2, MH, D), jnp.float32),
            pltpu.VMEM((2, MH, D), jnp.float32),
            pltpu.SemaphoreType.DMA((4,)),
            pltpu.SemaphoreType.DMA((4,)),
            pltpu.SemaphoreType.DMA((2,)),
            pltpu.SemaphoreType.DMA((2,)),
        ],
        compiler_params=pltpu.CompilerParams(collective_id=0),
    )(partial, gamma2d)
